# Initial kernel scaffold; baseline (speedup 1.0000x reference)
import jax
import jax.numpy as jnp
from jax.experimental import pallas as pl


def kernel(feats, edge_index, W, b, a_l, a_r, eps):
    raise NotImplementedError("write your pallas kernel here")


# speedup vs baseline: 19.7063x; 19.7063x over previous
import functools

import jax
import jax.numpy as jnp
from jax import lax
from jax.experimental import pallas as pl
from jax.experimental.pallas import tpu as pltpu
from jax.experimental.pallas import tpu_sc as plsc

N = 10000
D = 128
E = 320000
DH = 64
DYH = 80
NC = 2
NS = 16
EPT = E // NS
BLK = 80
NBLK = EPT // BLK
ROWS_PT = N // NS
ZR = 125
ZB = 25
LANES = 16


def _prep_body(f_ref, al_ref, ar_ref, el_ref, er_ref, g_ref, mx_ref):
    i = pl.program_id(0)
    f = f_ref[...]
    el = jnp.sum(f * al_ref[...], axis=1)
    er = jnp.sum(f * ar_ref[...], axis=1)
    el_ref[...] = el[:, None]
    er_ref[...] = er[:, None]
    ml = jnp.max(el)
    mr = jnp.max(er)

    @pl.when(i == 0)
    def _():
        mx_ref[0] = ml
        mx_ref[1] = mr

    @pl.when(i > 0)
    def _():
        mx_ref[0] = jnp.maximum(mx_ref[0], ml)
        mx_ref[1] = jnp.maximum(mx_ref[1], mr)

    t = mx_ref[0] + mx_ref[1]
    g = jnp.where(t > 0.0, t, 0.2 * t)
    g_ref[...] = jnp.full((1, 128), g, jnp.float32)


_PREP_BL = 1000
_prep = pl.pallas_call(
    _prep_body,
    grid=(N // _PREP_BL,),
    in_specs=[
        pl.BlockSpec((_PREP_BL, D), lambda i: (i, 0)),
        pl.BlockSpec((1, D), lambda i: (0, 0)),
        pl.BlockSpec((1, D), lambda i: (0, 0)),
    ],
    out_specs=[
        pl.BlockSpec((_PREP_BL, 1), lambda i: (i, 0)),
        pl.BlockSpec((_PREP_BL, 1), lambda i: (i, 0)),
        pl.BlockSpec((1, 128), lambda i: (0, 0)),
    ],
    out_shape=[
        jax.ShapeDtypeStruct((N, 1), jnp.float32),
        jax.ShapeDtypeStruct((N, 1), jnp.float32),
        jax.ShapeDtypeStruct((1, 128), jnp.float32),
    ],
    scratch_shapes=[pltpu.SMEM((2,), jnp.float32)],
)


def _sc_body(y_hbm, el_hbm, er_hbm, src_hbm, dst_hbm, g_hbm, out_hbm,
             el_v, er_v, src_v, dst_v, eb_v, row_v, row_w, zero_v, g_v,
             sg0, sg1, ss0, ss1, u_sh):
    c = lax.axis_index("c")
    s = lax.axis_index("s")

    pltpu.sync_copy(el_hbm, el_v)
    pltpu.sync_copy(er_hbm, er_v)
    pltpu.sync_copy(src_hbm.at[s], src_v)
    pltpu.sync_copy(dst_hbm.at[s], dst_v)
    pltpu.sync_copy(g_hbm.at[0, pl.ds(0, LANES)], g_v)
    gvec = g_v[...]

    z16 = jnp.zeros((LANES,), jnp.float32)

    @pl.loop(0, ZB)
    def _(r):
        for cc in range(DYH // LANES):
            zero_v[r, pl.ds(cc * LANES, LANES)] = z16

    @pl.loop(0, ROWS_PT // ZB)
    def _(j):
        pltpu.sync_copy(zero_v, u_sh.at[pl.ds(s * ROWS_PT + j * ZB, ZB)])

    plsc.subcore_barrier()

    def _gather(b, buf, sem):
        pltpu.async_copy(y_hbm.at[c].at[src_v.at[b]], buf, sem)

    def _wait_gather(b, buf, sem):
        pltpu.make_async_copy(y_hbm.at[c].at[src_v.at[b]], buf, sem).wait()

    def _scatter(b, buf, sem):
        pltpu.async_copy(buf, u_sh.at[dst_v.at[b]], sem, add=True)

    def _wait_scatter(b, buf, sem):
        pltpu.make_async_copy(buf, u_sh.at[dst_v.at[b]], sem).wait()

    def _process(b, buf):
        @pl.loop(0, BLK // LANES)
        def _(k):
            srow = src_v[b, pl.ds(k * LANES, LANES)]
            drow = dst_v[b, pl.ds(k * LANES, LANES)]
            t = plsc.load_gather(el_v, [srow]) + plsc.load_gather(er_v, [drow])
            t = jnp.where(t > 0.0, t, 0.2 * t)
            eb_v[pl.ds(k * LANES, LANES)] = jnp.exp(t - gvec)

        @pl.loop(0, BLK // LANES)
        def _(k):
            e16 = eb_v[pl.ds(k * LANES, LANES)]

            @pl.loop(0, LANES)
            def _(j):
                dn = lax.GatherDimensionNumbers(
                    offset_dims=(), collapsed_slice_dims=(0,),
                    start_index_map=(0,))
                ev = lax.gather(e16, lax.broadcast(j, (LANES, 1)), dn,
                                slice_sizes=(1,),
                                mode=lax.GatherScatterMode.PROMISE_IN_BOUNDS)
                r = k * LANES + j
                for cc in range(DYH // LANES):
                    buf[r, pl.ds(cc * LANES, LANES)] = (
                        buf[r, pl.ds(cc * LANES, LANES)] * ev)

    _gather(0, row_v, sg0)

    @pl.loop(0, NBLK // 2)
    def _(i):
        b0 = 2 * i
        b1 = b0 + 1
        _gather(b1, row_w, sg1)
        _wait_gather(b0, row_v, sg0)
        _process(b0, row_v)
        _scatter(b0, row_v, ss0)
        _wait_gather(b1, row_w, sg1)
        _process(b1, row_w)
        _scatter(b1, row_w, ss1)
        _wait_scatter(b0, row_v, ss0)

        @pl.when(i < NBLK // 2 - 1)
        def _():
            _gather(b0 + 2, row_v, sg0)

        _wait_scatter(b1, row_w, ss1)

    plsc.subcore_barrier()

    @pl.loop(0, ROWS_PT // ZR)
    def _(j):
        base = s * ROWS_PT + j * ZR
        pltpu.sync_copy(u_sh.at[pl.ds(base, ZR)],
                        out_hbm.at[c, pl.ds(base, ZR)])


_sc_cp = pltpu.CompilerParams(
    needs_layout_passes=False, use_tc_tiling_on_sc=False)

_sc = functools.partial(
    pl.kernel,
    compiler_params=_sc_cp,
    out_type=jax.ShapeDtypeStruct((NC, N, DYH), jnp.float32),
    mesh=plsc.VectorSubcoreMesh(core_axis_name="c", subcore_axis_name="s"),
    scratch_types=[
        pltpu.VMEM((N,), jnp.float32),
        pltpu.VMEM((N,), jnp.float32),
        pltpu.VMEM((NBLK, BLK), jnp.int32),
        pltpu.VMEM((NBLK, BLK), jnp.int32),
        pltpu.VMEM((BLK,), jnp.float32),
        pltpu.VMEM((BLK, DYH), jnp.float32),
        pltpu.VMEM((BLK, DYH), jnp.float32),
        pltpu.VMEM((ZB, DYH), jnp.float32),
        pltpu.VMEM((LANES,), jnp.float32),
        pltpu.SemaphoreType.DMA,
        pltpu.SemaphoreType.DMA,
        pltpu.SemaphoreType.DMA,
        pltpu.SemaphoreType.DMA,
        pltpu.VMEM_SHARED((N, DYH), jnp.float32),
    ],
)(_sc_body)


def _final_body(ua_ref, ub_ref, f_ref, w_ref, b_ref, sc_ref, g_ref, o_ref):
    ua = ua_ref[...]
    ub = ub_ref[...]
    eps_term = 1e-9 * jnp.exp(-g_ref[0:1, 0:1])
    da = jnp.sum(ua[:, DH:DYH], axis=1, keepdims=True) + eps_term
    db = jnp.sum(ub[:, DH:DYH], axis=1, keepdims=True) + eps_term
    agg = jnp.concatenate([ua[:, :DH] / da, ub[:, :DH] / db], axis=1)
    h = f_ref[...] * sc_ref[...] + agg
    z = jnp.dot(h, w_ref[...], precision=lax.Precision.HIGHEST) + b_ref[...]
    o_ref[...] = jnp.where(z > 0.0, z, jnp.exp(jnp.minimum(z, 0.0)) - 1.0)


_FIN_BL = 1000
_final = pl.pallas_call(
    _final_body,
    grid=(N // _FIN_BL,),
    in_specs=[
        pl.BlockSpec((_FIN_BL, DYH), lambda i: (i, 0)),
        pl.BlockSpec((_FIN_BL, DYH), lambda i: (i, 0)),
        pl.BlockSpec((_FIN_BL, D), lambda i: (i, 0)),
        pl.BlockSpec((D, D), lambda i: (0, 0)),
        pl.BlockSpec((1, D), lambda i: (0, 0)),
        pl.BlockSpec((1, D), lambda i: (0, 0)),
        pl.BlockSpec((1, 128), lambda i: (0, 0)),
    ],
    out_specs=pl.BlockSpec((_FIN_BL, D), lambda i: (i, 0)),
    out_shape=jax.ShapeDtypeStruct((N, D), jnp.float32),
)


def kernel(feats, edge_index, W, b, a_l, a_r, eps):
    el, er, g = _prep(feats, a_l.reshape(1, D), a_r.reshape(1, D))
    pad = jnp.concatenate(
        [jnp.ones((N, 1), jnp.float32),
         jnp.zeros((N, DYH - DH - 1), jnp.float32)], axis=1)
    y = jnp.stack(
        [jnp.concatenate([feats[:, :DH], pad], axis=1),
         jnp.concatenate([feats[:, DH:], pad], axis=1)])
    src3 = edge_index[0].reshape(NS, NBLK, BLK)
    dst3 = edge_index[1].reshape(NS, NBLK, BLK)
    u2 = _sc(y, el.reshape(N), er.reshape(N), src3, dst3, g)
    scale = jnp.broadcast_to((1.0 + eps).astype(jnp.float32), (1, D))
    return _final(u2[0], u2[1], feats, W, b.reshape(1, D), scale, g)


# speedup vs baseline: 21.0112x; 1.0662x over previous
import functools

import jax
import jax.numpy as jnp
from jax import lax
from jax.experimental import pallas as pl
from jax.experimental.pallas import tpu as pltpu
from jax.experimental.pallas import tpu_sc as plsc

N = 10000
D = 128
E = 320000
DH = 64
DYH = 80
NC = 2
NS = 16
EPT = E // NS
BLK = 80
NBLK = EPT // BLK
ROWS_PT = N // NS
ZR = 125
ZB = 25
LANES = 16


def _prep_body(f_ref, al_ref, ar_ref, el_ref, er_ref, g_ref, y_ref, mx_ref):
    i = pl.program_id(0)
    f = f_ref[...]
    el = jnp.sum(f * al_ref[...], axis=1)
    er = jnp.sum(f * ar_ref[...], axis=1)
    el_ref[...] = el[:, None]
    er_ref[...] = er[:, None]
    ones = jnp.ones((f.shape[0], 1), jnp.float32)
    zeros = jnp.zeros((f.shape[0], DYH - DH - 1), jnp.float32)
    y_ref[0] = jnp.concatenate([f[:, :DH], ones, zeros], axis=1)
    y_ref[1] = jnp.concatenate([f[:, DH:], ones, zeros], axis=1)
    ml = jnp.max(el)
    mr = jnp.max(er)

    @pl.when(i == 0)
    def _():
        mx_ref[0] = ml
        mx_ref[1] = mr

    @pl.when(i > 0)
    def _():
        mx_ref[0] = jnp.maximum(mx_ref[0], ml)
        mx_ref[1] = jnp.maximum(mx_ref[1], mr)

    t = mx_ref[0] + mx_ref[1]
    g = jnp.where(t > 0.0, t, 0.2 * t)
    g_ref[...] = jnp.full((1, 128), g, jnp.float32)


_PREP_BL = 1000
_prep = pl.pallas_call(
    _prep_body,
    grid=(N // _PREP_BL,),
    in_specs=[
        pl.BlockSpec((_PREP_BL, D), lambda i: (i, 0)),
        pl.BlockSpec((1, D), lambda i: (0, 0)),
        pl.BlockSpec((1, D), lambda i: (0, 0)),
    ],
    out_specs=[
        pl.BlockSpec((_PREP_BL, 1), lambda i: (i, 0)),
        pl.BlockSpec((_PREP_BL, 1), lambda i: (i, 0)),
        pl.BlockSpec((1, 128), lambda i: (0, 0)),
        pl.BlockSpec((NC, _PREP_BL, DYH), lambda i: (0, i, 0)),
    ],
    out_shape=[
        jax.ShapeDtypeStruct((N, 1), jnp.float32),
        jax.ShapeDtypeStruct((N, 1), jnp.float32),
        jax.ShapeDtypeStruct((1, 128), jnp.float32),
        jax.ShapeDtypeStruct((NC, N, DYH), jnp.float32),
    ],
    scratch_shapes=[pltpu.SMEM((2,), jnp.float32)],
)


def _sc_body(y_hbm, el_hbm, er_hbm, src_hbm, dst_hbm, g_hbm, out_hbm,
             el_v, er_v, src_v, dst_v, eb_v, row_v, row_w, zero_v, g_v,
             sg0, sg1, ss0, ss1, u_sh):
    c = lax.axis_index("c")
    s = lax.axis_index("s")

    pltpu.sync_copy(el_hbm, el_v)
    pltpu.sync_copy(er_hbm, er_v)
    pltpu.sync_copy(src_hbm.at[s], src_v)
    pltpu.sync_copy(dst_hbm.at[s], dst_v)
    pltpu.sync_copy(g_hbm.at[0, pl.ds(0, LANES)], g_v)
    gvec = g_v[...]

    z16 = jnp.zeros((LANES,), jnp.float32)

    @pl.loop(0, ZB)
    def _(r):
        for cc in range(DYH // LANES):
            zero_v[r, pl.ds(cc * LANES, LANES)] = z16

    @pl.loop(0, ROWS_PT // ZB)
    def _(j):
        pltpu.sync_copy(zero_v, u_sh.at[pl.ds(s * ROWS_PT + j * ZB, ZB)])

    plsc.subcore_barrier()

    def _gather(b, buf, sem):
        pltpu.async_copy(y_hbm.at[c].at[src_v.at[b]], buf, sem)

    def _wait_gather(b, buf, sem):
        pltpu.make_async_copy(y_hbm.at[c].at[src_v.at[b]], buf, sem).wait()

    def _scatter(b, buf, sem):
        pltpu.async_copy(buf, u_sh.at[dst_v.at[b]], sem, add=True)

    def _wait_scatter(b, buf, sem):
        pltpu.make_async_copy(buf, u_sh.at[dst_v.at[b]], sem).wait()

    def _process(b, buf):
        @pl.loop(0, BLK // LANES)
        def _(k):
            srow = src_v[b, pl.ds(k * LANES, LANES)]
            drow = dst_v[b, pl.ds(k * LANES, LANES)]
            t = plsc.load_gather(el_v, [srow]) + plsc.load_gather(er_v, [drow])
            t = jnp.where(t > 0.0, t, 0.2 * t)
            eb_v[pl.ds(k * LANES, LANES)] = jnp.exp(t - gvec)

        @plsc.parallel_loop(0, BLK // LANES)
        def _(k):
            e16 = eb_v[pl.ds(k * LANES, LANES)]

            @plsc.parallel_loop(0, LANES)
            def _(j):
                dn = lax.GatherDimensionNumbers(
                    offset_dims=(), collapsed_slice_dims=(0,),
                    start_index_map=(0,))
                ev = lax.gather(e16, lax.broadcast(j, (LANES, 1)), dn,
                                slice_sizes=(1,),
                                mode=lax.GatherScatterMode.PROMISE_IN_BOUNDS)
                r = k * LANES + j
                for cc in range(DYH // LANES):
                    buf[r, pl.ds(cc * LANES, LANES)] = (
                        buf[r, pl.ds(cc * LANES, LANES)] * ev)

    _gather(0, row_v, sg0)

    @pl.loop(0, NBLK // 2)
    def _(i):
        b0 = 2 * i
        b1 = b0 + 1
        _gather(b1, row_w, sg1)
        _wait_gather(b0, row_v, sg0)
        _process(b0, row_v)
        _scatter(b0, row_v, ss0)
        _wait_gather(b1, row_w, sg1)
        _process(b1, row_w)
        _scatter(b1, row_w, ss1)
        _wait_scatter(b0, row_v, ss0)

        @pl.when(i < NBLK // 2 - 1)
        def _():
            _gather(b0 + 2, row_v, sg0)

        _wait_scatter(b1, row_w, ss1)

    plsc.subcore_barrier()

    @pl.loop(0, ROWS_PT // ZR)
    def _(j):
        base = s * ROWS_PT + j * ZR
        pltpu.sync_copy(u_sh.at[pl.ds(base, ZR)],
                        out_hbm.at[c, pl.ds(base, ZR)])


_sc_cp = pltpu.CompilerParams(
    needs_layout_passes=False, use_tc_tiling_on_sc=False)

_sc = functools.partial(
    pl.kernel,
    compiler_params=_sc_cp,
    out_type=jax.ShapeDtypeStruct((NC, N, DYH), jnp.float32),
    mesh=plsc.VectorSubcoreMesh(core_axis_name="c", subcore_axis_name="s"),
    scratch_types=[
        pltpu.VMEM((N,), jnp.float32),
        pltpu.VMEM((N,), jnp.float32),
        pltpu.VMEM((NBLK, BLK), jnp.int32),
        pltpu.VMEM((NBLK, BLK), jnp.int32),
        pltpu.VMEM((BLK,), jnp.float32),
        pltpu.VMEM((BLK, DYH), jnp.float32),
        pltpu.VMEM((BLK, DYH), jnp.float32),
        pltpu.VMEM((ZB, DYH), jnp.float32),
        pltpu.VMEM((LANES,), jnp.float32),
        pltpu.SemaphoreType.DMA,
        pltpu.SemaphoreType.DMA,
        pltpu.SemaphoreType.DMA,
        pltpu.SemaphoreType.DMA,
        pltpu.VMEM_SHARED((N, DYH), jnp.float32),
    ],
)(_sc_body)


def _final_body(ua_ref, ub_ref, f_ref, w_ref, b_ref, sc_ref, g_ref, o_ref):
    ua = ua_ref[...]
    ub = ub_ref[...]
    eps_term = 1e-9 * jnp.exp(-g_ref[0:1, 0:1])
    da = jnp.sum(ua[:, DH:DYH], axis=1, keepdims=True) + eps_term
    db = jnp.sum(ub[:, DH:DYH], axis=1, keepdims=True) + eps_term
    agg = jnp.concatenate([ua[:, :DH] / da, ub[:, :DH] / db], axis=1)
    h = f_ref[...] * sc_ref[...] + agg
    z = jnp.dot(h, w_ref[...], precision=lax.Precision.HIGHEST) + b_ref[...]
    o_ref[...] = jnp.where(z > 0.0, z, jnp.exp(jnp.minimum(z, 0.0)) - 1.0)


_FIN_BL = 1000
_final = pl.pallas_call(
    _final_body,
    grid=(N // _FIN_BL,),
    in_specs=[
        pl.BlockSpec((_FIN_BL, DYH), lambda i: (i, 0)),
        pl.BlockSpec((_FIN_BL, DYH), lambda i: (i, 0)),
        pl.BlockSpec((_FIN_BL, D), lambda i: (i, 0)),
        pl.BlockSpec((D, D), lambda i: (0, 0)),
        pl.BlockSpec((1, D), lambda i: (0, 0)),
        pl.BlockSpec((1, D), lambda i: (0, 0)),
        pl.BlockSpec((1, 128), lambda i: (0, 0)),
    ],
    out_specs=pl.BlockSpec((_FIN_BL, D), lambda i: (i, 0)),
    out_shape=jax.ShapeDtypeStruct((N, D), jnp.float32),
)


def kernel(feats, edge_index, W, b, a_l, a_r, eps):
    el, er, g, y = _prep(feats, a_l.reshape(1, D), a_r.reshape(1, D))
    src3 = edge_index[0].reshape(NS, NBLK, BLK)
    dst3 = edge_index[1].reshape(NS, NBLK, BLK)
    u2 = _sc(y, el.reshape(N), er.reshape(N), src3, dst3, g)
    scale = jnp.broadcast_to((1.0 + eps).astype(jnp.float32), (1, D))
    return _final(u2[0], u2[1], feats, W, b.reshape(1, D), scale, g)


# speedup vs baseline: 21.5769x; 1.0269x over previous
import functools

import jax
import jax.numpy as jnp
from jax import lax
from jax.experimental import pallas as pl
from jax.experimental.pallas import tpu as pltpu
from jax.experimental.pallas import tpu_sc as plsc

N = 10000
D = 128
E = 320000
DH = 64
DYH = 80
NC = 2
NS = 16
EPT = E // NS
BLK = 80
NBLK = EPT // BLK
ROWS_PT = N // NS
ZR = 125
ZB = 25
LANES = 16


def _prep_body(f_ref, al_ref, ar_ref, el_ref, er_ref, g_ref, y_ref, mx_ref):
    i = pl.program_id(0)
    f = f_ref[...]
    el = jnp.sum(f * al_ref[...], axis=1)
    er = jnp.sum(f * ar_ref[...], axis=1)
    el_ref[...] = el[:, None]
    er_ref[...] = er[:, None]
    ones = jnp.ones((f.shape[0], 1), jnp.float32)
    zeros = jnp.zeros((f.shape[0], DYH - DH - 1), jnp.float32)
    y_ref[0] = jnp.concatenate([f[:, :DH], ones, zeros], axis=1)
    y_ref[1] = jnp.concatenate([f[:, DH:], ones, zeros], axis=1)
    ml = jnp.max(el)
    mr = jnp.max(er)

    @pl.when(i == 0)
    def _():
        mx_ref[0] = ml
        mx_ref[1] = mr

    @pl.when(i > 0)
    def _():
        mx_ref[0] = jnp.maximum(mx_ref[0], ml)
        mx_ref[1] = jnp.maximum(mx_ref[1], mr)

    t = mx_ref[0] + mx_ref[1]
    g = jnp.where(t > 0.0, t, 0.2 * t)
    g_ref[...] = jnp.full((1, 128), g, jnp.float32)


_PREP_BL = 1000
_prep = pl.pallas_call(
    _prep_body,
    grid=(N // _PREP_BL,),
    in_specs=[
        pl.BlockSpec((_PREP_BL, D), lambda i: (i, 0)),
        pl.BlockSpec((1, D), lambda i: (0, 0)),
        pl.BlockSpec((1, D), lambda i: (0, 0)),
    ],
    out_specs=[
        pl.BlockSpec((_PREP_BL, 1), lambda i: (i, 0)),
        pl.BlockSpec((_PREP_BL, 1), lambda i: (i, 0)),
        pl.BlockSpec((1, 128), lambda i: (0, 0)),
        pl.BlockSpec((NC, _PREP_BL, DYH), lambda i: (0, i, 0)),
    ],
    out_shape=[
        jax.ShapeDtypeStruct((N, 1), jnp.float32),
        jax.ShapeDtypeStruct((N, 1), jnp.float32),
        jax.ShapeDtypeStruct((1, 128), jnp.float32),
        jax.ShapeDtypeStruct((NC, N, DYH), jnp.float32),
    ],
    scratch_shapes=[pltpu.SMEM((2,), jnp.float32)],
)


def _sc_body(y_hbm, el_hbm, er_hbm, src_hbm, dst_hbm, g_hbm, out_hbm,
             el_v, er_v, src_v, dst_v, eb_v, row_v, row_w, zero_v, g_v,
             sg0, sg1, ss0, ss1, u_sh):
    c = lax.axis_index("c")
    s = lax.axis_index("s")

    pltpu.sync_copy(el_hbm, el_v)
    pltpu.sync_copy(er_hbm, er_v)
    pltpu.sync_copy(src_hbm.at[s], src_v)
    pltpu.sync_copy(dst_hbm.at[s], dst_v)
    pltpu.sync_copy(g_hbm.at[0, pl.ds(0, LANES)], g_v)
    gvec = g_v[...]

    z16 = jnp.zeros((LANES,), jnp.float32)

    @pl.loop(0, ZB)
    def _(r):
        for cc in range(DYH // LANES):
            zero_v[r, pl.ds(cc * LANES, LANES)] = z16

    @pl.loop(0, ROWS_PT // ZB)
    def _(j):
        pltpu.sync_copy(zero_v, u_sh.at[pl.ds(s * ROWS_PT + j * ZB, ZB)])

    plsc.subcore_barrier()

    def _gather(b, buf, sem):
        pltpu.async_copy(y_hbm.at[c].at[src_v.at[b]], buf, sem)

    def _wait_gather(b, buf, sem):
        pltpu.make_async_copy(y_hbm.at[c].at[src_v.at[b]], buf, sem).wait()

    def _scatter(b, buf, sem):
        pltpu.async_copy(buf, u_sh.at[dst_v.at[b]], sem, add=True)

    def _wait_scatter(b, buf, sem):
        pltpu.make_async_copy(buf, u_sh.at[dst_v.at[b]], sem).wait()

    def _process(b, buf):
        @plsc.parallel_loop(0, BLK // LANES)
        def _(k):
            srow = src_v[b, pl.ds(k * LANES, LANES)]
            drow = dst_v[b, pl.ds(k * LANES, LANES)]
            t = plsc.load_gather(el_v, [srow]) + plsc.load_gather(er_v, [drow])
            t = jnp.where(t > 0.0, t, 0.2 * t)
            eb_v[pl.ds(k * LANES, LANES)] = jnp.exp(t - gvec)

        @plsc.parallel_loop(0, BLK // LANES)
        def _(k):
            e16 = eb_v[pl.ds(k * LANES, LANES)]

            @plsc.parallel_loop(0, LANES)
            def _(j):
                dn = lax.GatherDimensionNumbers(
                    offset_dims=(), collapsed_slice_dims=(0,),
                    start_index_map=(0,))
                ev = lax.gather(e16, lax.broadcast(j, (LANES, 1)), dn,
                                slice_sizes=(1,),
                                mode=lax.GatherScatterMode.PROMISE_IN_BOUNDS)
                r = k * LANES + j
                for cc in range(DYH // LANES):
                    buf[r, pl.ds(cc * LANES, LANES)] = (
                        buf[r, pl.ds(cc * LANES, LANES)] * ev)

    _gather(0, row_v, sg0)

    @pl.loop(0, NBLK // 2)
    def _(i):
        b0 = 2 * i
        b1 = b0 + 1
        _gather(b1, row_w, sg1)
        _wait_gather(b0, row_v, sg0)
        _process(b0, row_v)
        _scatter(b0, row_v, ss0)
        _wait_gather(b1, row_w, sg1)
        _process(b1, row_w)
        _scatter(b1, row_w, ss1)
        _wait_scatter(b0, row_v, ss0)

        @pl.when(i < NBLK // 2 - 1)
        def _():
            _gather(b0 + 2, row_v, sg0)

        _wait_scatter(b1, row_w, ss1)

    plsc.subcore_barrier()

    @pl.loop(0, ROWS_PT // ZR)
    def _(j):
        base = s * ROWS_PT + j * ZR
        pltpu.sync_copy(u_sh.at[pl.ds(base, ZR)],
                        out_hbm.at[c, pl.ds(base, ZR)])


_sc_cp = pltpu.CompilerParams(
    needs_layout_passes=False, use_tc_tiling_on_sc=False)

_sc = functools.partial(
    pl.kernel,
    compiler_params=_sc_cp,
    out_type=jax.ShapeDtypeStruct((NC, N, DYH), jnp.float32),
    mesh=plsc.VectorSubcoreMesh(core_axis_name="c", subcore_axis_name="s"),
    scratch_types=[
        pltpu.VMEM((N,), jnp.float32),
        pltpu.VMEM((N,), jnp.float32),
        pltpu.VMEM((NBLK, BLK), jnp.int32),
        pltpu.VMEM((NBLK, BLK), jnp.int32),
        pltpu.VMEM((BLK,), jnp.float32),
        pltpu.VMEM((BLK, DYH), jnp.float32),
        pltpu.VMEM((BLK, DYH), jnp.float32),
        pltpu.VMEM((ZB, DYH), jnp.float32),
        pltpu.VMEM((LANES,), jnp.float32),
        pltpu.SemaphoreType.DMA,
        pltpu.SemaphoreType.DMA,
        pltpu.SemaphoreType.DMA,
        pltpu.SemaphoreType.DMA,
        pltpu.VMEM_SHARED((N, DYH), jnp.float32),
    ],
)(_sc_body)


def _final_body(ua_ref, ub_ref, f_ref, w_ref, b_ref, sc_ref, g_ref, o_ref):
    ua = ua_ref[...]
    ub = ub_ref[...]
    eps_term = 1e-9 * jnp.exp(-g_ref[0:1, 0:1])
    da = jnp.sum(ua[:, DH:DYH], axis=1, keepdims=True) + eps_term
    db = jnp.sum(ub[:, DH:DYH], axis=1, keepdims=True) + eps_term
    agg = jnp.concatenate([ua[:, :DH] / da, ub[:, :DH] / db], axis=1)
    h = f_ref[...] * sc_ref[...] + agg
    z = jnp.dot(h, w_ref[...], precision=lax.Precision.HIGHEST) + b_ref[...]
    o_ref[...] = jnp.where(z > 0.0, z, jnp.exp(jnp.minimum(z, 0.0)) - 1.0)


_FIN_BL = 1000
_final = pl.pallas_call(
    _final_body,
    grid=(N // _FIN_BL,),
    in_specs=[
        pl.BlockSpec((_FIN_BL, DYH), lambda i: (i, 0)),
        pl.BlockSpec((_FIN_BL, DYH), lambda i: (i, 0)),
        pl.BlockSpec((_FIN_BL, D), lambda i: (i, 0)),
        pl.BlockSpec((D, D), lambda i: (0, 0)),
        pl.BlockSpec((1, D), lambda i: (0, 0)),
        pl.BlockSpec((1, D), lambda i: (0, 0)),
        pl.BlockSpec((1, 128), lambda i: (0, 0)),
    ],
    out_specs=pl.BlockSpec((_FIN_BL, D), lambda i: (i, 0)),
    out_shape=jax.ShapeDtypeStruct((N, D), jnp.float32),
)


def kernel(feats, edge_index, W, b, a_l, a_r, eps):
    el, er, g, y = _prep(feats, a_l.reshape(1, D), a_r.reshape(1, D))
    src3 = edge_index[0].reshape(NS, NBLK, BLK)
    dst3 = edge_index[1].reshape(NS, NBLK, BLK)
    u2 = _sc(y, el.reshape(N), er.reshape(N), src3, dst3, g)
    scale = jnp.broadcast_to((1.0 + eps).astype(jnp.float32), (1, D))
    return _final(u2[0], u2[1], feats, W, b.reshape(1, D), scale, g)
